# baseline (speedup 1.0000x reference)
import functools

import numpy as np
import jax
import jax.numpy as jnp
from jax import lax
from jax.experimental import pallas as pl
from jax.experimental.pallas import tpu as pltpu
from jax.experimental.pallas import tpu_sc as plsc

N = 10000
E = 320000
D = 128
G = 128
L_LAYERS = 3

NC = 2
NS = 16
CH = 80
LANES = 16

NCHUNKS = E // CH
NW = NC * NS
CPT = NCHUNKS // NW

UNITS_UNIFORM = NS * 78


def _edge_body(bf16, z_hbm, src_hbm, dst_hbm, ew_hbm, out_hbm, agg_sh,
               srcv0, srcv1, dstv0, dstv1, dstv2, dstv3, zr0, zr1, ew0, ew1,
               sem_i0, sem_i1, sem_g0, sem_g1, sem_e0, sem_e1,
               sem_s0, sem_s1):
    c = lax.axis_index("c")
    s = lax.axis_index("s")
    wid = c * NS + s
    start = wid * CPT

    srcv = (srcv0, srcv1)
    dstv = (dstv0, dstv1, dstv2, dstv3)
    zrs = (zr0, zr1)
    ews = (ew0, ew1)
    sem_i = (sem_i0, sem_i1)
    sem_g = (sem_g0, sem_g1)
    sem_e = (sem_e0, sem_e1)
    sem_s = (sem_s0, sem_s1)

    def idx_issue(gci, b, d):
        pltpu.async_copy(src_hbm.at[pl.ds(gci * CH, CH)], srcv[b], sem_i[b])
        pltpu.async_copy(dst_hbm.at[pl.ds(gci * CH, CH)], dstv[d], sem_i[b])

    def idx_wait(gci, b, d):
        pltpu.make_async_copy(src_hbm.at[pl.ds(gci * CH, CH)], srcv[b], sem_i[b]).wait()
        pltpu.make_async_copy(dst_hbm.at[pl.ds(gci * CH, CH)], dstv[d], sem_i[b]).wait()

    def data_issue(gci, b):
        pltpu.async_copy(z_hbm.at[srcv[b]], zrs[b], sem_g[b])
        pltpu.async_copy(ew_hbm.at[pl.ds(gci * CH, CH)], ews[b], sem_e[b])

    def data_wait(gci, b):
        pltpu.make_async_copy(z_hbm.at[srcv[b]], zrs[b], sem_g[b]).wait()
        pltpu.make_async_copy(ew_hbm.at[pl.ds(gci * CH, CH)], ews[b], sem_e[b]).wait()

    msg = zrs if bf16 else ews

    def scatter_issue(b, d):
        pltpu.async_copy(msg[b], agg_sh.at[dstv[d]], sem_s[b], add=True)

    def scatter_wait(b, d):
        pltpu.make_async_copy(msg[b], agg_sh.at[dstv[d]], sem_s[b]).wait()

    def compute(b):
        if bf16:
            def _row(r, _):
                for cc in range(D // 32):
                    wv = ews[b][r, pl.ds(16 * cc, 16)]
                    e_ev = lax.bitcast_convert_type(wv << 16, jnp.float32)
                    e_od = lax.bitcast_convert_type(
                        wv & jnp.int32(-65536), jnp.float32)
                    sl_ev = pl.ds(32 * cc, LANES)
                    sl_od = pl.ds(32 * cc + LANES, LANES)
                    zrs[b][r, sl_ev] = jnp.maximum(zrs[b][r, sl_ev] + e_ev, 0.0)
                    zrs[b][r, sl_od] = jnp.maximum(zrs[b][r, sl_od] + e_od, 0.0)
                return 0
        else:
            def _row(r, _):
                for cc in range(D // LANES):
                    sl = pl.ds(cc * LANES, LANES)
                    ews[b][r, sl] = jnp.maximum(zrs[b][r, sl] + ews[b][r, sl], 0.0)
                return 0
        lax.fori_loop(0, CH, _row, 0)

    idx_issue(start, 0, 0)
    idx_issue(start + 1, 1, 1)

    def _zero_row(r, _):
        for cc in range(D // LANES):
            zrs[1][r, pl.ds(cc * LANES, LANES)] = jnp.zeros((LANES,), jnp.float32)
        return 0
    lax.fori_loop(0, CH, _zero_row, 0)

    row0 = s * 624
    for k in range(7):
        pltpu.sync_copy(zrs[1].at[pl.ds(0, CH)],
                        agg_sh.at[pl.ds(row0 + k * CH, CH)])
    pltpu.sync_copy(zrs[1].at[pl.ds(0, 64)],
                    agg_sh.at[pl.ds(row0 + 7 * CH, 64)])

    @pl.when(s < 2)
    def _zero_tail():
        pltpu.sync_copy(zrs[1].at[pl.ds(0, 8)],
                        agg_sh.at[pl.ds((UNITS_UNIFORM + s) * 8, 8)])

    idx_wait(start, 0, 0)
    data_issue(start, 0)
    plsc.subcore_barrier()

    def _quad(j, _):
        for t in range(4):
            k = 4 * j + t
            b = t % 2
            gci = start + k
            data_wait(gci, b)

            if t == 0:
                @pl.when(j > 0)
                def _():
                    scatter_wait(1 - b, 3)
            else:
                scatter_wait(1 - b, (t - 1) % 4)
            idx_wait(gci + 1, 1 - b, (t + 1) % 4)
            data_issue(gci + 1, 1 - b)

            compute(b)
            scatter_issue(b, t)

            if t < 3:
                idx_issue(gci + 2, b, (t + 2) % 4)
            else:
                @pl.when(j < CPT // 4 - 1)
                def _():
                    idx_issue(gci + 2, b, (t + 2) % 4)
        return 0
    lax.fori_loop(0, CPT // 4, _quad, 0)

    gci_last = start + CPT - 1
    data_wait(gci_last, 0)
    scatter_wait(1, 3)
    compute(0)
    scatter_issue(0, 0)
    scatter_wait(0, 0)

    plsc.subcore_barrier()

    for k in range(7):
        pltpu.sync_copy(agg_sh.at[pl.ds(row0 + k * CH, CH)],
                        out_hbm.at[c, pl.ds(row0 + k * CH, CH)])
    pltpu.sync_copy(agg_sh.at[pl.ds(row0 + 7 * CH, 64)],
                    out_hbm.at[c, pl.ds(row0 + 7 * CH, 64)])

    @pl.when(s < 2)
    def _dump_tail():
        pltpu.sync_copy(agg_sh.at[pl.ds((UNITS_UNIFORM + s) * 8, 8)],
                        out_hbm.at[c, pl.ds((UNITS_UNIFORM + s) * 8, 8)])


@functools.lru_cache(maxsize=None)
def _make_edge_kernel(bf16):
    ew_slab = (pltpu.VMEM((CH, D // 2), jnp.int32) if bf16
               else pltpu.VMEM((CH, D), jnp.float32))
    return functools.partial(
        pl.kernel,
        out_type=jax.ShapeDtypeStruct((NC, N, D), jnp.float32),
        mesh=plsc.VectorSubcoreMesh(core_axis_name="c", subcore_axis_name="s",
                                    num_cores=NC, num_subcores=NS),
        scratch_types=[
            pltpu.VMEM_SHARED((N, D), jnp.float32),
            pltpu.VMEM((CH,), jnp.int32),
            pltpu.VMEM((CH,), jnp.int32),
            pltpu.VMEM((CH,), jnp.int32),
            pltpu.VMEM((CH,), jnp.int32),
            pltpu.VMEM((CH,), jnp.int32),
            pltpu.VMEM((CH,), jnp.int32),
            pltpu.VMEM((CH, D), jnp.float32),
            pltpu.VMEM((CH, D), jnp.float32),
            ew_slab,
            ew_slab,
            pltpu.SemaphoreType.DMA,
            pltpu.SemaphoreType.DMA,
            pltpu.SemaphoreType.DMA,
            pltpu.SemaphoreType.DMA,
            pltpu.SemaphoreType.DMA,
            pltpu.SemaphoreType.DMA,
            pltpu.SemaphoreType.DMA,
            pltpu.SemaphoreType.DMA,
        ],
    )(functools.partial(_edge_body, bf16))


BR = 5000
NB = N // BR
BN_EPS = 1e-5


def _dense_body(z_ref, a0_ref, a1_ref, bt_ref, W1_ref, W1a_ref, b1_ref, g_ref,
                be_ref, W2_ref, b2_ref, P_ref, zout_ref, zperm_ref, gout_ref,
                h1s, ssum, ssq, pools, cnts):
    p = pl.program_id(0)
    b = pl.program_id(1)

    @pl.when(p == 0)
    def _phase0():
        h1 = (jnp.dot(z_ref[...], W1_ref[...], preferred_element_type=jnp.float32)
              + jnp.dot(a0_ref[...] + a1_ref[...], W1a_ref[...],
                        preferred_element_type=jnp.float32)
              + b1_ref[...])
        h1s[pl.ds(b * BR, BR), :] = h1
        srow = jnp.sum(h1, axis=0, keepdims=True)
        qrow = jnp.sum(h1 * h1, axis=0, keepdims=True)

        @pl.when(b == 0)
        def _():
            ssum[...] = srow
            ssq[...] = qrow

        @pl.when(b > 0)
        def _():
            ssum[...] += srow
            ssq[...] += qrow

    @pl.when(p == 1)
    def _phase1():
        mu = ssum[...] / N
        var = ssq[...] / N - mu * mu
        h1 = h1s[pl.ds(b * BR, BR), :]
        y = (h1 - mu) * (g_ref[...] * lax.rsqrt(var + BN_EPS)) + be_ref[...]
        y = jnp.maximum(y, 0.0)
        h2 = jnp.dot(y, W2_ref[...], preferred_element_type=jnp.float32) + b2_ref[...]
        zo = jnp.maximum(h2, 0.0)
        zout_ref[...] = zo
        zperm_ref[...] = jnp.dot(zo, P_ref[...], preferred_element_type=jnp.float32)

        gi = lax.broadcasted_iota(jnp.int32, (G, BR), 0)
        mask = (bt_ref[0] == gi).astype(jnp.float32)
        pool = lax.dot_general(mask, zo, (((1,), (0,)), ((), ())),
                               preferred_element_type=jnp.float32)
        cnt = jnp.sum(mask, axis=1, keepdims=True)

        @pl.when(b == 0)
        def _():
            pools[...] = pool
            cnts[...] = cnt

        @pl.when(b > 0)
        def _():
            pools[...] += pool
            cnts[...] += cnt

        @pl.when(b == NB - 1)
        def _():
            gout_ref[...] = pools[...] / jnp.maximum(cnts[...], 1.0)


_dense_kernel = pl.pallas_call(
    _dense_body,
    grid=(2, NB),
    in_specs=[
        pl.BlockSpec((BR, D), lambda p, b: (jnp.where(p == 0, b, NB - 1), 0)),
        pl.BlockSpec((BR, D), lambda p, b: (jnp.where(p == 0, b, NB - 1), 0)),
        pl.BlockSpec((BR, D), lambda p, b: (jnp.where(p == 0, b, NB - 1), 0)),
        pl.BlockSpec((1, 1, BR), lambda p, b: (b, 0, 0)),
        pl.BlockSpec((D, D), lambda p, b: (0, 0)),
        pl.BlockSpec((D, D), lambda p, b: (0, 0)),
        pl.BlockSpec((1, D), lambda p, b: (0, 0)),
        pl.BlockSpec((1, D), lambda p, b: (0, 0)),
        pl.BlockSpec((1, D), lambda p, b: (0, 0)),
        pl.BlockSpec((D, D), lambda p, b: (0, 0)),
        pl.BlockSpec((1, D), lambda p, b: (0, 0)),
        pl.BlockSpec((D, D), lambda p, b: (0, 0)),
    ],
    out_specs=[
        pl.BlockSpec((BR, D), lambda p, b: (b, 0)),
        pl.BlockSpec((BR, D), lambda p, b: (b, 0)),
        pl.BlockSpec((G, D), lambda p, b: (0, 0)),
    ],
    out_shape=[
        jax.ShapeDtypeStruct((N, D), jnp.float32),
        jax.ShapeDtypeStruct((N, D), jnp.float32),
        jax.ShapeDtypeStruct((G, D), jnp.float32),
    ],
    scratch_shapes=[
        pltpu.VMEM((N, D), jnp.float32),
        pltpu.VMEM((1, D), jnp.float32),
        pltpu.VMEM((1, D), jnp.float32),
        pltpu.VMEM((G, D), jnp.float32),
        pltpu.VMEM((G, 1), jnp.float32),
    ],
)


_PERM = np.concatenate(
    [np.concatenate([32 * c + 2 * np.arange(16), 32 * c + 2 * np.arange(16) + 1])
     for c in range(D // 32)]).astype(np.int32)
_PMAT = np.zeros((D, D), np.float32)
_PMAT[_PERM, np.arange(D)] = 1.0


def kernel(x, edge_index, edge_weights, batch, W1, b1, gamma, beta, W2, b2):
    src = edge_index[0]
    dst = edge_index[1]
    batch3d = batch.reshape(NB, 1, BR)
    ew_i32 = lax.bitcast_convert_type(
        edge_weights.astype(jnp.bfloat16).reshape(E, D // 2, 2), jnp.int32)
    P = jnp.asarray(_PMAT)
    z = x
    zperm = None
    gs = []
    edge_nat = _make_edge_kernel(False)
    edge_bf16 = _make_edge_kernel(True)
    for l in range(L_LAYERS):
        if l == 0:
            agg = edge_nat(z, src, dst, edge_weights)
            W1a = W1[l]
        else:
            agg = edge_bf16(zperm, src, dst, ew_i32)
            W1a = W1[l][_PERM]
        z, zperm, g_l = _dense_kernel(z, agg[0], agg[1], batch3d,
                                      W1[l], W1a, b1[l].reshape(1, D),
                                      gamma[l].reshape(1, D),
                                      beta[l].reshape(1, D), W2[l],
                                      b2[l].reshape(1, D), P)
        gs.append(g_l)
    return z, jnp.concatenate(gs, axis=1)


# speedup vs baseline: 2.2102x; 2.2102x over previous
import functools

import jax
import jax.numpy as jnp
from jax import lax
from jax.experimental import pallas as pl
from jax.experimental.pallas import tpu as pltpu
from jax.experimental.pallas import tpu_sc as plsc

N = 10000
E = 320000
D = 128
G = 128
L_LAYERS = 3

NC = 2
NS = 16
CH = 80
LANES = 16

NCHUNKS = E // CH
NW = NC * NS
CPT = NCHUNKS // NW

UNITS_UNIFORM = NS * 78


def _edge_body(z_hbm, src_hbm, dst_hbm, ew_hbm, out_hbm, agg_sh,
               srcv0, srcv1, dstv0, dstv1, dstv2, dstv3, zr0, zr1, ew0, ew1,
               sem_i0, sem_i1, sem_g0, sem_g1, sem_e0, sem_e1,
               sem_s0, sem_s1):
    c = lax.axis_index("c")
    s = lax.axis_index("s")
    wid = c * NS + s
    start = wid * CPT

    srcv = (srcv0, srcv1)
    dstv = (dstv0, dstv1, dstv2, dstv3)
    zrs = (zr0, zr1)
    ews = (ew0, ew1)
    sem_i = (sem_i0, sem_i1)
    sem_g = (sem_g0, sem_g1)
    sem_e = (sem_e0, sem_e1)
    sem_s = (sem_s0, sem_s1)

    def idx_issue(gci, b, d):
        pltpu.async_copy(src_hbm.at[pl.ds(gci * CH, CH)], srcv[b], sem_i[b])
        pltpu.async_copy(dst_hbm.at[pl.ds(gci * CH, CH)], dstv[d], sem_i[b])

    def idx_wait(gci, b, d):
        pltpu.make_async_copy(src_hbm.at[pl.ds(gci * CH, CH)], srcv[b], sem_i[b]).wait()
        pltpu.make_async_copy(dst_hbm.at[pl.ds(gci * CH, CH)], dstv[d], sem_i[b]).wait()

    def data_issue(gci, b):
        pltpu.async_copy(z_hbm.at[srcv[b]], zrs[b], sem_g[b])
        pltpu.async_copy(ew_hbm.at[pl.ds(gci * CH, CH)], ews[b], sem_e[b])

    def data_wait(gci, b):
        pltpu.make_async_copy(z_hbm.at[srcv[b]], zrs[b], sem_g[b]).wait()
        pltpu.make_async_copy(ew_hbm.at[pl.ds(gci * CH, CH)], ews[b], sem_e[b]).wait()

    def scatter_issue(b, d):
        pltpu.async_copy(ews[b], agg_sh.at[dstv[d]], sem_s[b], add=True)

    def scatter_wait(b, d):
        pltpu.make_async_copy(ews[b], agg_sh.at[dstv[d]], sem_s[b]).wait()

    def compute(b):
        def _row(r, _):
            for cc in range(D // LANES):
                sl = pl.ds(cc * LANES, LANES)
                ews[b][r, sl] = jnp.maximum(zrs[b][r, sl] + ews[b][r, sl], 0.0)
            return 0
        lax.fori_loop(0, CH, _row, 0)

    idx_issue(start, 0, 0)
    idx_issue(start + 1, 1, 1)

    def _zero_row(r, _):
        for cc in range(D // LANES):
            zrs[1][r, pl.ds(cc * LANES, LANES)] = jnp.zeros((LANES,), jnp.float32)
        return 0
    lax.fori_loop(0, CH, _zero_row, 0)

    row0 = s * 624
    for k in range(7):
        pltpu.sync_copy(zrs[1].at[pl.ds(0, CH)],
                        agg_sh.at[pl.ds(row0 + k * CH, CH)])
    pltpu.sync_copy(zrs[1].at[pl.ds(0, 64)],
                    agg_sh.at[pl.ds(row0 + 7 * CH, 64)])

    @pl.when(s < 2)
    def _zero_tail():
        pltpu.sync_copy(zrs[1].at[pl.ds(0, 8)],
                        agg_sh.at[pl.ds((UNITS_UNIFORM + s) * 8, 8)])

    idx_wait(start, 0, 0)
    data_issue(start, 0)
    plsc.subcore_barrier()

    def _quad(j, _):
        for t in range(4):
            k = 4 * j + t
            b = t % 2
            gci = start + k
            data_wait(gci, b)

            if t == 0:
                @pl.when(j > 0)
                def _():
                    scatter_wait(1 - b, 3)
            else:
                scatter_wait(1 - b, (t - 1) % 4)
            idx_wait(gci + 1, 1 - b, (t + 1) % 4)
            data_issue(gci + 1, 1 - b)

            compute(b)
            scatter_issue(b, t)

            if t < 3:
                idx_issue(gci + 2, b, (t + 2) % 4)
            else:
                @pl.when(j < CPT // 4 - 1)
                def _():
                    idx_issue(gci + 2, b, (t + 2) % 4)
        return 0
    lax.fori_loop(0, CPT // 4, _quad, 0)

    gci_last = start + CPT - 1
    data_wait(gci_last, 0)
    scatter_wait(1, 3)
    compute(0)
    scatter_issue(0, 0)
    scatter_wait(0, 0)

    plsc.subcore_barrier()

    for k in range(7):
        pltpu.sync_copy(agg_sh.at[pl.ds(row0 + k * CH, CH)],
                        out_hbm.at[c, pl.ds(row0 + k * CH, CH)])
    pltpu.sync_copy(agg_sh.at[pl.ds(row0 + 7 * CH, 64)],
                    out_hbm.at[c, pl.ds(row0 + 7 * CH, 64)])

    @pl.when(s < 2)
    def _dump_tail():
        pltpu.sync_copy(agg_sh.at[pl.ds((UNITS_UNIFORM + s) * 8, 8)],
                        out_hbm.at[c, pl.ds((UNITS_UNIFORM + s) * 8, 8)])


@functools.lru_cache(maxsize=None)
def _make_edge_kernel():
    return functools.partial(
        pl.kernel,
        out_type=jax.ShapeDtypeStruct((NC, N, D), jnp.float32),
        mesh=plsc.VectorSubcoreMesh(core_axis_name="c", subcore_axis_name="s",
                                    num_cores=NC, num_subcores=NS),
        scratch_types=[
            pltpu.VMEM_SHARED((N, D), jnp.float32),
            pltpu.VMEM((CH,), jnp.int32),
            pltpu.VMEM((CH,), jnp.int32),
            pltpu.VMEM((CH,), jnp.int32),
            pltpu.VMEM((CH,), jnp.int32),
            pltpu.VMEM((CH,), jnp.int32),
            pltpu.VMEM((CH,), jnp.int32),
            pltpu.VMEM((CH, D), jnp.float32),
            pltpu.VMEM((CH, D), jnp.float32),
            pltpu.VMEM((CH, D), jnp.float32),
            pltpu.VMEM((CH, D), jnp.float32),
            pltpu.SemaphoreType.DMA,
            pltpu.SemaphoreType.DMA,
            pltpu.SemaphoreType.DMA,
            pltpu.SemaphoreType.DMA,
            pltpu.SemaphoreType.DMA,
            pltpu.SemaphoreType.DMA,
            pltpu.SemaphoreType.DMA,
            pltpu.SemaphoreType.DMA,
        ],
    )(_edge_body)


BR = 5000
NB = N // BR
BN_EPS = 1e-5


def _dense_body(z_ref, a0_ref, a1_ref, bt_ref, W1_ref, b1_ref, g_ref, be_ref,
                W2_ref, b2_ref, zout_ref, gout_ref,
                h1s, ssum, ssq, pools, cnts):
    p = pl.program_id(0)
    b = pl.program_id(1)

    @pl.when(p == 0)
    def _phase0():
        u = z_ref[...] + a0_ref[...] + a1_ref[...]
        h1 = jnp.dot(u, W1_ref[...], preferred_element_type=jnp.float32) + b1_ref[...]
        h1s[pl.ds(b * BR, BR), :] = h1
        srow = jnp.sum(h1, axis=0, keepdims=True)
        qrow = jnp.sum(h1 * h1, axis=0, keepdims=True)

        @pl.when(b == 0)
        def _():
            ssum[...] = srow
            ssq[...] = qrow

        @pl.when(b > 0)
        def _():
            ssum[...] += srow
            ssq[...] += qrow

    @pl.when(p == 1)
    def _phase1():
        mu = ssum[...] / N
        var = ssq[...] / N - mu * mu
        h1 = h1s[pl.ds(b * BR, BR), :]
        y = (h1 - mu) * (g_ref[...] * lax.rsqrt(var + BN_EPS)) + be_ref[...]
        y = jnp.maximum(y, 0.0)
        h2 = jnp.dot(y, W2_ref[...], preferred_element_type=jnp.float32) + b2_ref[...]
        zo = jnp.maximum(h2, 0.0)
        zout_ref[...] = zo

        gi = lax.broadcasted_iota(jnp.int32, (G, BR), 0)
        mask = (bt_ref[0] == gi).astype(jnp.float32)
        pool = lax.dot_general(mask, zo, (((1,), (0,)), ((), ())),
                               preferred_element_type=jnp.float32)
        cnt = jnp.sum(mask, axis=1, keepdims=True)

        @pl.when(b == 0)
        def _():
            pools[...] = pool
            cnts[...] = cnt

        @pl.when(b > 0)
        def _():
            pools[...] += pool
            cnts[...] += cnt

        @pl.when(b == NB - 1)
        def _():
            gout_ref[...] = pools[...] / jnp.maximum(cnts[...], 1.0)


_dense_kernel = pl.pallas_call(
    _dense_body,
    grid=(2, NB),
    in_specs=[
        pl.BlockSpec((BR, D), lambda p, b: (jnp.where(p == 0, b, NB - 1), 0)),
        pl.BlockSpec((BR, D), lambda p, b: (jnp.where(p == 0, b, NB - 1), 0)),
        pl.BlockSpec((BR, D), lambda p, b: (jnp.where(p == 0, b, NB - 1), 0)),
        pl.BlockSpec((1, 1, BR), lambda p, b: (b, 0, 0)),
        pl.BlockSpec((D, D), lambda p, b: (0, 0)),
        pl.BlockSpec((1, D), lambda p, b: (0, 0)),
        pl.BlockSpec((1, D), lambda p, b: (0, 0)),
        pl.BlockSpec((1, D), lambda p, b: (0, 0)),
        pl.BlockSpec((D, D), lambda p, b: (0, 0)),
        pl.BlockSpec((1, D), lambda p, b: (0, 0)),
    ],
    out_specs=[
        pl.BlockSpec((BR, D), lambda p, b: (b, 0)),
        pl.BlockSpec((G, D), lambda p, b: (0, 0)),
    ],
    out_shape=[
        jax.ShapeDtypeStruct((N, D), jnp.float32),
        jax.ShapeDtypeStruct((G, D), jnp.float32),
    ],
    scratch_shapes=[
        pltpu.VMEM((N, D), jnp.float32),
        pltpu.VMEM((1, D), jnp.float32),
        pltpu.VMEM((1, D), jnp.float32),
        pltpu.VMEM((G, D), jnp.float32),
        pltpu.VMEM((G, 1), jnp.float32),
    ],
)


def kernel(x, edge_index, edge_weights, batch, W1, b1, gamma, beta, W2, b2):
    src = edge_index[0]
    dst = edge_index[1]
    batch3d = batch.reshape(NB, 1, BR)
    z = x
    gs = []
    edge_kernel = _make_edge_kernel()
    for l in range(L_LAYERS):
        agg = edge_kernel(z, src, dst, edge_weights)
        z, g_l = _dense_kernel(z, agg[0], agg[1], batch3d,
                               W1[l], b1[l].reshape(1, D), gamma[l].reshape(1, D),
                               beta[l].reshape(1, D), W2[l], b2[l].reshape(1, D))
        gs.append(g_l)
    return z, jnp.concatenate(gs, axis=1)
